# Initial kernel scaffold; baseline (speedup 1.0000x reference)
import jax
import jax.numpy as jnp
from jax.experimental import pallas as pl


def kernel(h, edge_index, We1, be1, We2, be2, Wn1, bn1, Wn2, bn2):
    raise NotImplementedError("write your pallas kernel here")


# speedup vs baseline: 1.7907x; 1.7907x over previous
import functools

import jax
import jax.numpy as jnp
from jax import lax
from jax.experimental import pallas as pl
from jax.experimental.pallas import tpu as pltpu
from jax.experimental.pallas import tpu_sc as plsc

NC = 2
NS = 16
NW = NC * NS
LANES = 16

CH = 80


def _sc_mesh():
  return plsc.VectorSubcoreMesh(
      core_axis_name="c", subcore_axis_name="s", num_cores=NC, num_subcores=NS)


def _pre_body(h_ref, w1t_ref, w1b_ref, be1_ref, wn1t_ref, bn1_ref,
              a_ref, b_ref, c_ref):
  h = h_ref[...]
  a_ref[...] = jnp.dot(h, w1t_ref[...], preferred_element_type=jnp.float32)
  b_ref[...] = (jnp.dot(h, w1b_ref[...], preferred_element_type=jnp.float32)
                + be1_ref[...])
  c_ref[...] = (jnp.dot(h, wn1t_ref[...], preferred_element_type=jnp.float32)
                + bn1_ref[...])


def _tc_pre(h, w1t, w1b, be1, wn1t, bn1, block=1000):
  n, d = h.shape
  hh = w1t.shape[1]
  grid = n // block
  full = lambda s: pl.BlockSpec(s, lambda i: (0, 0))
  blk = pl.BlockSpec((block, d), lambda i: (i, 0))
  blk_o = pl.BlockSpec((block, hh), lambda i: (i, 0))
  return pl.pallas_call(
      _pre_body,
      grid=(grid,),
      in_specs=[blk, full((d, hh)), full((d, hh)), full((1, hh)),
                full((d, hh)), full((1, hh))],
      out_specs=[blk_o, blk_o, blk_o],
      out_shape=[jax.ShapeDtypeStruct((n, hh), jnp.float32)] * 3,
  )(h, w1t, w1b, be1, wn1t, bn1)


def _gather_body(a_hbm, b_hbm, row_hbm, col_hbm, g_hbm,
                 idxr_v, idxc_v, bufa_v, bufb_v, sema, semb, *, nch, hh):
  cid = lax.axis_index("c")
  sid = lax.axis_index("s")
  wid = sid * NC + cid
  pltpu.sync_copy(row_hbm.at[wid], idxr_v)
  pltpu.sync_copy(col_hbm.at[wid], idxc_v)
  ebase = wid * nch * CH

  def chunk(i, carry):
    cpa = pltpu.async_copy(a_hbm.at[idxr_v.at[i]], bufa_v, sema)
    cpb = pltpu.async_copy(b_hbm.at[idxc_v.at[i]], bufb_v, semb)
    cpa.wait()
    cpb.wait()

    def rowfn(r, carry2):
      for j in range(hh // LANES):
        sl = pl.ds(j * LANES, LANES)
        bufa_v[r, sl] = jnp.maximum(bufa_v[r, sl] + bufb_v[r, sl], 0.0)
      return carry2

    lax.fori_loop(0, CH, rowfn, 0, unroll=2)
    pltpu.sync_copy(bufa_v, g_hbm.at[pl.ds(ebase + i * CH, CH)])
    return carry

  lax.fori_loop(0, nch, chunk, 0)
  pltpu.sync_copy(bufa_v, g_hbm.at[pl.ds(ebase + (nch - 1) * CH, CH)])
  for t in range(4):
    pltpu.sync_copy(g_hbm.at[pl.ds(ebase + (nch - 1) * CH + t * 8, 8)],
                    bufb_v.at[pl.ds(t * 8, 8)])
  plsc.subcore_barrier()


def _sc_gather(a, b, row2, col2, e, hh):
  nch = row2.shape[1]
  kern = pl.kernel(
      functools.partial(_gather_body, nch=nch, hh=hh),
      out_type=jax.ShapeDtypeStruct((e, hh), jnp.float32),
      mesh=_sc_mesh(),
      scratch_types=[
          pltpu.VMEM((nch, CH), jnp.int32),
          pltpu.VMEM((nch, CH), jnp.int32),
          pltpu.VMEM((CH, hh), jnp.float32),
          pltpu.VMEM((CH, hh), jnp.float32),
          pltpu.SemaphoreType.DMA,
          pltpu.SemaphoreType.DMA,
      ],
  )
  return kern(a, b, row2, col2)


def _edge_body(g_ref, w_ref, be_ref, m_ref):
  acc = jnp.dot(g_ref[...], w_ref[...], preferred_element_type=jnp.float32)
  m_ref[...] = jnp.maximum(acc * 0.01 + be_ref[...] * 0.01, 0.0)


def _tc_edge(g, w2, be2, block=2000):
  e, hh = g.shape
  blk = pl.BlockSpec((block, hh), lambda i: (i, 0))
  return pl.pallas_call(
      _edge_body,
      grid=(e // block,),
      in_specs=[blk, pl.BlockSpec((hh, hh), lambda i: (0, 0)),
                pl.BlockSpec((1, hh), lambda i: (0, 0))],
      out_specs=blk,
      out_shape=jax.ShapeDtypeStruct((e, hh), jnp.float32),
  )(g, w2, be2)


def _scatter_body(m_hbm, row_hbm, out_hbm, idxr_v, buf_v, z_v, idxz_v,
                  agg_sh, *, nch, npad, hh):
  cid = lax.axis_index("c")
  sid = lax.axis_index("s")
  wid = sid * NC + cid
  rows_per_sub = npad // NS
  zrows = z_v.shape[0]

  def zrow(r, c):
    for j in range(hh // LANES):
      z_v[r, pl.ds(j * LANES, LANES)] = jnp.zeros((LANES,), jnp.float32)
    return c
  lax.fori_loop(0, zrows, zrow, 0)
  nzc = rows_per_sub // zrows
  for t in range(nzc + 1):
    if t == nzc:
      base = jnp.int32(npad - zrows)
    else:
      base = (sid * rows_per_sub + t * zrows).astype(jnp.int32)
    for j in range(zrows // LANES):
      idxz_v[t, pl.ds(j * LANES, LANES)] = (
          base + j * LANES + lax.iota(jnp.int32, LANES))
  for t in range(nzc):
    pltpu.sync_copy(z_v, agg_sh.at[idxz_v.at[t]])
  def spin(i, c):
    idxz_v[nzc, pl.ds(0, LANES)] = (jnp.int32(npad - CH)
                                    + lax.iota(jnp.int32, LANES))
    return c
  lax.fori_loop(0, 12000, spin, 0)
  plsc.subcore_barrier()

  pltpu.sync_copy(row_hbm.at[wid], idxr_v)
  ebase = wid * nch * CH

  def chunk(i, carry):
    pltpu.sync_copy(m_hbm.at[pl.ds(ebase + i * CH, CH)], buf_v)
    pltpu.sync_copy(buf_v, agg_sh.at[idxr_v.at[i]], add=True)
    return carry

  lax.fori_loop(0, nch, chunk, 0)
  def spin2(i, c):
    idxz_v[nzc, pl.ds(0, LANES)] = (jnp.int32(npad - CH)
                                    + lax.iota(jnp.int32, LANES))
    return c
  lax.fori_loop(0, 6000, spin2, 0)
  plsc.subcore_barrier()

  nwc = rows_per_sub // 160
  for t in list(range(nwc)) + list(reversed(range(nwc))) + list(range(nwc)):
    off = sid * rows_per_sub + t * 160
    pltpu.sync_copy(agg_sh.at[pl.ds(off, 160)],
                    out_hbm.at[cid, pl.ds(off, 160)])
  for t in range(4):
    pltpu.sync_copy(
        out_hbm.at[cid, pl.ds(sid * rows_per_sub + rows_per_sub - 32 + t * 8,
                              8)],
        buf_v.at[pl.ds(t * 8, 8)])
  plsc.subcore_barrier()


def _sc_scatter(m, row2, npad, hh):
  nch = row2.shape[1]
  kern = pl.kernel(
      functools.partial(_scatter_body, nch=nch, npad=npad, hh=hh),
      out_type=jax.ShapeDtypeStruct((NC, npad, hh), jnp.float32),
      mesh=_sc_mesh(),
      scratch_types=[
          pltpu.VMEM((nch, CH), jnp.int32),
          pltpu.VMEM((CH, hh), jnp.float32),
          pltpu.VMEM((CH, hh), jnp.float32),
          pltpu.VMEM((npad // NS // CH + 1, CH), jnp.int32),
          pltpu.VMEM_SHARED((npad, hh), jnp.float32),
      ],
  )
  return kern(m, row2)


def _node_body(h_ref, c_ref, p0_ref, p1_ref, wn1b_ref, wn2_ref, bn2_ref,
               o_ref):
  agg = p0_ref[...] + p1_ref[...]
  u = jnp.maximum(
      c_ref[...] + jnp.dot(agg, wn1b_ref[...],
                           preferred_element_type=jnp.float32), 0.0)
  o_ref[...] = (h_ref[...]
                + jnp.dot(u, wn2_ref[...], preferred_element_type=jnp.float32)
                + bn2_ref[...])


def _tc_node(h, c, p0, p1, wn1b, wn2, bn2, block=1000):
  n, d = h.shape
  hh = wn1b.shape[0]
  blk_h = pl.BlockSpec((block, hh), lambda i: (i, 0))
  blk_d = pl.BlockSpec((block, d), lambda i: (i, 0))
  full = lambda s: pl.BlockSpec(s, lambda i: (0, 0))
  return pl.pallas_call(
      _node_body,
      grid=(n // block,),
      in_specs=[blk_d, blk_h, blk_h, blk_h, full((hh, hh)), full((hh, d)),
                full((1, d))],
      out_specs=blk_d,
      out_shape=jax.ShapeDtypeStruct((n, d), jnp.float32),
  )(h, c, p0, p1, wn1b, wn2, bn2)


def kernel(h, edge_index, We1, be1, We2, be2, Wn1, bn1, Wn2, bn2):
  n, d = h.shape
  hh = We1.shape[1]
  e = edge_index.shape[1]
  row = edge_index[0]
  col = edge_index[1]
  row2 = row.reshape(NW, e // (NW * CH), CH)
  col2 = col.reshape(NW, e // (NW * CH), CH)
  npad = ((n + NS * 8 - 1) // (NS * 8)) * (NS * 8)

  a, b, c = _tc_pre(h, We1[:d], We1[d:], be1.reshape(1, hh),
                    Wn1[:d], bn1.reshape(1, hh))
  g = _sc_gather(a, b, row2, col2, e, hh)
  m = _tc_edge(g, We2, be2.reshape(1, hh))
  agg = jax.ops.segment_sum(m, row, num_segments=n)
  zero = jnp.zeros_like(agg)
  return _tc_node(h, c, agg, zero, Wn1[d:], Wn2, bn2.reshape(1, d))
